# Initial kernel scaffold; baseline (speedup 1.0000x reference)
import jax
import jax.numpy as jnp
from jax.experimental import pallas as pl


def kernel(x, edge_index, batch, train, Wl1, Wr1, att1, b1, g1, be1, Wl2, Wr2, att2, b2, g2, be2, Wl3, Wr3, att3, b3, g3, be3, Wl4, Wr4, att4, b4, g4, be4, lin1_W, lin1_b, g5, be5, lin2_W, lin2_b):
    raise NotImplementedError("write your pallas kernel here")


# speedup vs baseline: 2.0896x; 2.0896x over previous
import jax
import jax.numpy as jnp
from jax.experimental import pallas as pl

NG = 64


def _gatv2(x, src, dst, Wl, Wr, att, b):
    n = x.shape[0]
    xl = x @ Wl
    xr = x @ Wr
    e = jax.nn.leaky_relu(xl[src] + xr[dst], 0.2) @ att
    ex = jnp.exp(e)
    den = jax.ops.segment_sum(ex, dst, num_segments=n)
    num = jax.ops.segment_sum(ex[:, None] * xl[src], dst, num_segments=n)
    return num / (den[:, None] + 1e-16) + b


def _bn(h, g, b):
    mu = jnp.mean(h, axis=0)
    var = jnp.var(h, axis=0)
    return (h - mu) / jnp.sqrt(var + 1e-5) * g + b


def _head_body(hcat_ref, w1_ref, b1_ref, g5_ref, be5_ref, w2_ref, b2_ref,
               sig_ref, lin_ref):
    h = jnp.maximum(hcat_ref[...] @ w1_ref[...] + b1_ref[...], 0.0)
    mu = jnp.mean(h, axis=0, keepdims=True)
    var = jnp.mean((h - mu) ** 2, axis=0, keepdims=True)
    h = (h - mu) / jnp.sqrt(var + 1e-5) * g5_ref[...] + be5_ref[...]
    o = h @ w2_ref[...] + b2_ref[...]
    lin_ref[...] = o
    sig_ref[...] = jax.nn.sigmoid(o)


def _head(hcat, lin1_W, lin1_b, g5, be5, lin2_W, lin2_b):
    dout = lin2_W.shape[1]
    return pl.pallas_call(
        _head_body,
        out_shape=(
            jax.ShapeDtypeStruct((NG, dout), jnp.float32),
            jax.ShapeDtypeStruct((NG, dout), jnp.float32),
        ),
    )(hcat, lin1_W, lin1_b[None, :], g5[None, :], be5[None, :],
      lin2_W, lin2_b[None, :])


def kernel(x, edge_index, batch, train, Wl1, Wr1, att1, b1, g1, be1, Wl2, Wr2,
           att2, b2, g2, be2, Wl3, Wr3, att3, b3, g3, be3, Wl4, Wr4, att4, b4,
           g4, be4, lin1_W, lin1_b, g5, be5, lin2_W, lin2_b):
    n = x.shape[0]
    loop = jnp.arange(n, dtype=edge_index.dtype)
    src = jnp.concatenate([edge_index[0], loop])
    dst = jnp.concatenate([edge_index[1], loop])
    h1 = _bn(jax.nn.relu(_gatv2(x, src, dst, Wl1, Wr1, att1, b1)), g1, be1)
    h2 = _bn(jax.nn.relu(_gatv2(h1, src, dst, Wl2, Wr2, att2, b2)), g2, be2)
    h3 = _bn(jax.nn.relu(_gatv2(h2, src, dst, Wl3, Wr3, att3, b3)), g3, be3)
    h4 = _bn(jax.nn.relu(_gatv2(h3, src, dst, Wl4, Wr4, att4, b4)), g4, be4)
    p1 = jax.ops.segment_sum(h1, batch, num_segments=NG)
    p2 = jax.ops.segment_sum(h2, batch, num_segments=NG)
    p3 = jax.ops.segment_sum(h3, batch, num_segments=NG)
    p4 = jax.ops.segment_sum(h4, batch, num_segments=NG)
    hcat = jnp.concatenate([p1, p2, p3, p4], axis=1)
    sig, lin = _head(hcat, lin1_W, lin1_b, g5, be5, lin2_W, lin2_b)
    return (sig, lin)


# speedup vs baseline: 3.2531x; 1.5568x over previous
import functools

import jax
import jax.numpy as jnp
from jax import lax
from jax.experimental import pallas as pl
from jax.experimental.pallas import tpu as pltpu
from jax.experimental.pallas import tpu_sc as plsc

N = 10000
NG = 64
E0 = 320000
EP = 335872
EB = EP // 128
H = N // 2

_GDN = lax.GatherDimensionNumbers(
    offset_dims=(), collapsed_slice_dims=(0,), start_index_map=(0,))
_MESH = plsc.VectorSubcoreMesh(core_axis_name="c", subcore_axis_name="s")


def _iota16():
    return lax.iota(jnp.int32, 16)


def _hsum_all(a):
    for sh in (8, 4, 2, 1):
        idx = (_iota16() + sh) & 15
        a = a + lax.gather(a, idx[:, None], _GDN, (1,),
                           mode=lax.GatherScatterMode.PROMISE_IN_BOUNDS)
    return a


def _exp16(x):
    y = x * jnp.float32(1.0 / 16.0)
    p = jnp.float32(1.0 / 362880.0)
    for c in (1.0 / 40320.0, 1.0 / 5040.0, 1.0 / 720.0, 1.0 / 120.0,
              1.0 / 24.0, 1.0 / 6.0, 0.5, 1.0, 1.0):
        p = p * y + jnp.float32(c)
    for _ in range(4):
        p = p * p
    return p


def _rb16(x):
    c = x * jnp.float32(65537.0)
    return c - (c - x)


def _bcast_lane(a, j):
    return lax.gather(a, jnp.full((16, 1), j, jnp.int32), _GDN, (1,),
                      mode=lax.GatherScatterMode.PROMISE_IN_BOUNDS)


def _zero_rows(ref, rows, width):
    @pl.loop(0, rows)
    def _(r):
        for ci in range(width // 16):
            ref[r, pl.ds(ci * 16, 16)] = jnp.zeros((16,), jnp.float32)


def _row_chunks(rpt, src_rows):
    chunks = []
    off = 0
    while off < rpt:
        sz = min(src_rows, rpt - off)
        chunks.append((off, sz))
        off += sz
    return chunks


def _init_table(table, zsrc, s, rpt, chunks):
    for off, sz in chunks:
        pltpu.sync_copy(zsrc.at[pl.ds(0, sz)],
                        table.at[pl.ds(s * rpt + off, sz)])


def _copy_out(table, out_hbm, c, s, rpt, chunks):
    for off, sz in chunks:
        pltpu.sync_copy(table.at[pl.ds(s * rpt + off, sz)],
                        out_hbm.at[c, pl.ds(s * rpt + off, sz)])


def _scatter_rows(numst, table, didx, semS):
    descs = []
    for g in range(8):
        iv = didx[0, pl.ds(g * 16, 16)]
        descs.append(pltpu.async_copy(
            numst.at[pl.ds(g * 16, 16)], table.at[iv], semS, add=True))
    for dsc in descs:
        dsc.wait()


def _sc1a_body(xlh, xrh, src_hbm, dloc_hbm, dgat_hbm, att_hbm, out_hbm, w_hbm,
               table, sidx, didx, dgidx, AZ, denst, wrow, attv, semG, semS):
    c = lax.axis_index("c")
    s = lax.axis_index("s")
    rpt = 320
    chunks = _row_chunks(rpt, 128)
    _zero_rows(denst, 128, 128)
    _init_table(table, denst, s, rpt, chunks)
    pltpu.sync_copy(att_hbm, attv)
    plsc.subcore_barrier()

    @pl.loop(0, 164)
    def _(blk):
        row0 = s * 164 + blk
        pltpu.sync_copy(src_hbm.at[pl.ds(row0, 1)], sidx)
        pltpu.sync_copy(dloc_hbm.at[c, pl.ds(row0, 1)], didx)
        pltpu.sync_copy(dgat_hbm.at[pl.ds(row0, 1)], dgidx)
        descs = []
        for hh in range(2):
            descs.append(pltpu.async_copy(
                xlh.at[hh].at[sidx.at[0]],
                AZ.at[pl.ds(0, 128), pl.ds(hh * 128, 128)], semG))
            descs.append(pltpu.async_copy(
                xrh.at[hh].at[dgidx.at[0]],
                AZ.at[pl.ds(0, 128), pl.ds(256 + hh * 128, 128)], semG))
        for dsc in descs:
            dsc.wait()

        @pl.loop(0, 128, step=16)
        def _(i0):
            g16 = i0
            wlane = jnp.zeros((16,), jnp.float32)
            for j in range(16):
                acc = jnp.zeros((16,), jnp.float32)
                for ci in range(16):
                    av = attv[pl.ds(ci * 16, 16)]
                    z = (AZ[i0 + j, pl.ds(ci * 16, 16)]
                         + AZ[i0 + j, pl.ds(256 + ci * 16, 16)])
                    acc = acc + av * _rb16(jnp.maximum(z, 0.2 * z))
                w = _exp16(_hsum_all(acc))
                wlane = jnp.where(_iota16() == j, w, wlane)
                denst[i0 + j, pl.ds(0, 16)] = jnp.where(_iota16() == 0, w, 0.0)
            wrow[0, pl.ds(g16, 16)] = wlane

        _scatter_rows(denst, table, didx, semS)

        @pl.when(c == 0)
        def _():
            pltpu.sync_copy(wrow, w_hbm.at[pl.ds(row0, 1)])

    plsc.subcore_barrier()
    _copy_out(table, out_hbm, c, s, rpt, chunks)


_SC1A = pl.kernel(
    _sc1a_body,
    out_type=(jax.ShapeDtypeStruct((2, 5120, 128), jnp.float32),
              jax.ShapeDtypeStruct((EB, 128), jnp.float32)),
    mesh=_MESH,
    scratch_types=[
        pltpu.VMEM_SHARED((5120, 128), jnp.float32),
        pltpu.VMEM((1, 128), jnp.int32),
        pltpu.VMEM((1, 128), jnp.int32),
        pltpu.VMEM((1, 128), jnp.int32),
        pltpu.VMEM((128, 512), jnp.float32),
        pltpu.VMEM((128, 128), jnp.float32),
        pltpu.VMEM((1, 128), jnp.float32),
        pltpu.VMEM((256,), jnp.float32),
        pltpu.SemaphoreType.DMA,
        pltpu.SemaphoreType.DMA,
    ],
)


def _sc1b_body(xlh, src_hbm, dloc_hbm, w_hbm, out_hbm,
               table, sidx, didx, numst, wrow, semG, semS):
    c = lax.axis_index("c")
    s = lax.axis_index("s")
    rpt = 640
    chunks = _row_chunks(rpt, 128)
    _zero_rows(numst, 128, 128)
    _init_table(table, numst, s, rpt, chunks)
    plsc.subcore_barrier()

    @pl.loop(0, 164)
    def _(blk):
        row0 = s * 164 + blk
        pltpu.sync_copy(src_hbm.at[pl.ds(row0, 1)], sidx)
        pltpu.sync_copy(dloc_hbm.at[c, pl.ds(row0, 1)], didx)
        pltpu.sync_copy(w_hbm.at[pl.ds(row0, 1)], wrow)

        @pl.when(c == 0)
        def _():
            pltpu.async_copy(xlh.at[0].at[sidx.at[0]], numst, semG).wait()

        @pl.when(c == 1)
        def _():
            pltpu.async_copy(xlh.at[1].at[sidx.at[0]], numst, semG).wait()

        @pl.loop(0, 128, step=16)
        def _(i0):
            wv = wrow[0, pl.ds(i0, 16)]
            for j in range(16):
                w = _bcast_lane(wv, j)
                for ci in range(8):
                    numst[i0 + j, pl.ds(ci * 16, 16)] = (
                        w * numst[i0 + j, pl.ds(ci * 16, 16)])

        _scatter_rows(numst, table, didx, semS)

    plsc.subcore_barrier()
    _copy_out(table, out_hbm, c, s, rpt, chunks)


_SC1B = pl.kernel(
    _sc1b_body,
    out_type=jax.ShapeDtypeStruct((2, 10240, 128), jnp.float32),
    mesh=_MESH,
    scratch_types=[
        pltpu.VMEM_SHARED((10240, 128), jnp.float32),
        pltpu.VMEM((1, 128), jnp.int32),
        pltpu.VMEM((1, 128), jnp.int32),
        pltpu.VMEM((128, 128), jnp.float32),
        pltpu.VMEM((1, 128), jnp.float32),
        pltpu.SemaphoreType.DMA,
        pltpu.SemaphoreType.DMA,
    ],
)


def _sc2_body(xl_hbm, xr_hbm, src_hbm, dloc_hbm, dgat_hbm, att_hbm,
              outn_hbm, outd_hbm,
              tabn, tabd, sidx, didx, dgidx, numst, xrst, attv, semG, semS):
    c = lax.axis_index("c")
    s = lax.axis_index("s")
    rpt = 320
    chunks = _row_chunks(rpt, 128)
    _zero_rows(numst, 128, 128)
    _init_table(tabn, numst, s, rpt, chunks)
    _init_table(tabd, numst, s, rpt, chunks)
    pltpu.sync_copy(att_hbm, attv)
    plsc.subcore_barrier()

    @pl.loop(0, 164)
    def _(blk):
        row0 = s * 164 + blk
        pltpu.sync_copy(src_hbm.at[pl.ds(row0, 1)], sidx)
        pltpu.sync_copy(dloc_hbm.at[c, pl.ds(row0, 1)], didx)
        pltpu.sync_copy(dgat_hbm.at[pl.ds(row0, 1)], dgidx)
        d1 = pltpu.async_copy(xl_hbm.at[sidx.at[0]], numst, semG)
        d2 = pltpu.async_copy(xr_hbm.at[dgidx.at[0]], xrst, semG)
        d1.wait()
        d2.wait()

        @pl.loop(0, 128, step=16)
        def _(i0):
            for j in range(16):
                acc = jnp.zeros((16,), jnp.float32)
                for ci in range(8):
                    av = attv[pl.ds(ci * 16, 16)]
                    z = (numst[i0 + j, pl.ds(ci * 16, 16)]
                         + xrst[i0 + j, pl.ds(ci * 16, 16)])
                    acc = acc + av * _rb16(jnp.maximum(z, 0.2 * z))
                w = _exp16(_hsum_all(acc))
                xrst[i0 + j, pl.ds(0, 16)] = jnp.where(
                    _iota16() == 0, w, 0.0)
                for ci in range(8):
                    numst[i0 + j, pl.ds(ci * 16, 16)] = (
                        w * numst[i0 + j, pl.ds(ci * 16, 16)])

        _scatter_rows(numst, tabn, didx, semS)
        _scatter_rows(xrst, tabd, didx, semS)

    plsc.subcore_barrier()
    _copy_out(tabn, outn_hbm, c, s, rpt, chunks)
    _copy_out(tabd, outd_hbm, c, s, rpt, chunks)


_SC2 = pl.kernel(
    _sc2_body,
    out_type=(jax.ShapeDtypeStruct((2, 5120, 128), jnp.float32),
              jax.ShapeDtypeStruct((2, 5120, 128), jnp.float32)),
    mesh=_MESH,
    scratch_types=[
        pltpu.VMEM_SHARED((5120, 128), jnp.float32),
        pltpu.VMEM_SHARED((5120, 128), jnp.float32),
        pltpu.VMEM((1, 128), jnp.int32),
        pltpu.VMEM((1, 128), jnp.int32),
        pltpu.VMEM((1, 128), jnp.int32),
        pltpu.VMEM((128, 128), jnp.float32),
        pltpu.VMEM((128, 128), jnp.float32),
        pltpu.VMEM((128,), jnp.float32),
        pltpu.SemaphoreType.DMA,
        pltpu.SemaphoreType.DMA,
    ],
)


def _sc34_kernel(dreal):
    nsc = dreal // 16

    def body(xl_hbm, xr_hbm, src_hbm, dloc_hbm, dgat_hbm, att_hbm, out_hbm,
             table, sidx, didx, dgidx, A, numst, attv, semG, semS):
        c = lax.axis_index("c")
        s = lax.axis_index("s")
        rpt = 640
        chunks = _row_chunks(rpt, 128)
        _zero_rows(numst, 128, 128)
        _init_table(table, numst, s, rpt, chunks)
        pltpu.sync_copy(att_hbm, attv)
        plsc.subcore_barrier()

        @pl.loop(0, 82)
        def _(blk):
            row0 = (c * 16 + s) * 82 + blk
            pltpu.sync_copy(src_hbm.at[pl.ds(row0, 1)], sidx)
            pltpu.sync_copy(dloc_hbm.at[c, pl.ds(row0, 1)], didx)
            pltpu.sync_copy(dgat_hbm.at[pl.ds(row0, 1)], dgidx)
            d1 = pltpu.async_copy(xl_hbm.at[sidx.at[0]], A, semG)
            d2 = pltpu.async_copy(xr_hbm.at[dgidx.at[0]], numst, semG)
            d1.wait()
            d2.wait()

            @pl.loop(0, 128, step=16)
            def _(i0):
                for j in range(16):
                    acc = jnp.zeros((16,), jnp.float32)
                    for ci in range(nsc):
                        av = attv[pl.ds(ci * 16, 16)]
                        z = (A[i0 + j, pl.ds(ci * 16, 16)]
                             + numst[i0 + j, pl.ds(ci * 16, 16)])
                        acc = acc + av * _rb16(jnp.maximum(z, 0.2 * z))
                    w = _exp16(_hsum_all(acc))
                    numst[i0 + j, pl.ds(nsc * 16, 16)] = jnp.where(
                        _iota16() == 0, w, 0.0)
                    for ci in range(nsc):
                        numst[i0 + j, pl.ds(ci * 16, 16)] = (
                            w * A[i0 + j, pl.ds(ci * 16, 16)])

            _scatter_rows(numst, table, didx, semS)

        plsc.subcore_barrier()
        _copy_out(table, out_hbm, c, s, rpt, chunks)

    return pl.kernel(
        body,
        out_type=jax.ShapeDtypeStruct((2, 10240, 128), jnp.float32),
        mesh=_MESH,
        scratch_types=[
            pltpu.VMEM_SHARED((10240, 128), jnp.float32),
            pltpu.VMEM((1, 128), jnp.int32),
            pltpu.VMEM((1, 128), jnp.int32),
            pltpu.VMEM((1, 128), jnp.int32),
            pltpu.VMEM((128, 128), jnp.float32),
            pltpu.VMEM((128, 128), jnp.float32),
            pltpu.VMEM((128,), jnp.float32),
            pltpu.SemaphoreType.DMA,
            pltpu.SemaphoreType.DMA,
        ],
    )


_SC3 = _sc34_kernel(64)
_SC4 = _sc34_kernel(32)


def _mm2h_body(x_ref, wl_ref, wr_ref, xl_ref, xr_ref):
    xv = x_ref[...]
    xl = jnp.dot(xv, wl_ref[...], preferred_element_type=jnp.float32)
    xr = jnp.dot(xv, wr_ref[...], preferred_element_type=jnp.float32)
    xl_ref[...] = jnp.stack([xl[:, :128], xl[:, 128:]], axis=0)
    xr_ref[...] = jnp.stack([xr[:, :128], xr[:, 128:]], axis=0)


def _mm2h(x, Wl, Wr):
    n = x.shape[0]
    return pl.pallas_call(
        _mm2h_body,
        out_shape=(jax.ShapeDtypeStruct((2, n, 128), jnp.float32),
                   jax.ShapeDtypeStruct((2, n, 128), jnp.float32)),
    )(x, Wl, Wr)


def _bn(h, g, bet):
    mu = jnp.mean(h, axis=0, keepdims=True)
    var = jnp.mean((h - mu) ** 2, axis=0, keepdims=True)
    return (h - mu) * lax.rsqrt(var + 1e-5) * g + bet


def _hk1(tA, tB, b, g, bet):
    def body(tA_ref, tB_ref, b_ref, g_ref, be_ref, h_ref):
        num = jnp.concatenate([tB_ref[0, :N, :], tB_ref[1, :N, :]], axis=1)
        den = jnp.concatenate([tA_ref[0, :H, 0:1], tA_ref[1, :H, 0:1]],
                              axis=0)
        h = jnp.maximum(num / (den + 1e-16) + b_ref[...], 0.0)
        h_ref[...] = _bn(h, g_ref[...], be_ref[...])

    return pl.pallas_call(
        body, out_shape=jax.ShapeDtypeStruct((N, 256), jnp.float32),
    )(tA, tB, b[None, :], g[None, :], bet[None, :])


def _hk2(tn, td, b, g, bet):
    def body(tn_ref, td_ref, b_ref, g_ref, be_ref, h_ref):
        num = jnp.concatenate([tn_ref[0, :H, :], tn_ref[1, :H, :]], axis=0)
        den = jnp.concatenate([td_ref[0, :H, 0:1], td_ref[1, :H, 0:1]],
                              axis=0)
        h = jnp.maximum(num / (den + 1e-16) + b_ref[...], 0.0)
        h_ref[...] = _bn(h, g_ref[...], be_ref[...])

    return pl.pallas_call(
        body, out_shape=jax.ShapeDtypeStruct((N, 128), jnp.float32),
    )(tn, td, b[None, :], g[None, :], bet[None, :])


def _hk34(t, b, g, bet, d):
    def body(t_ref, b_ref, g_ref, be_ref, h_ref):
        num = t_ref[0, :N, :d] + t_ref[1, :N, :d]
        den = t_ref[0, :N, d:d + 1] + t_ref[1, :N, d:d + 1]
        h = jnp.maximum(num / (den + 1e-16) + b_ref[...], 0.0)
        h_ref[...] = _bn(h, g_ref[...], be_ref[...])

    return pl.pallas_call(
        body, out_shape=jax.ShapeDtypeStruct((N, d), jnp.float32),
    )(t, b[None, :], g[None, :], bet[None, :])


def _pool(h, batch2d):
    d = h.shape[1]

    def body(h_ref, batch_ref, p_ref):
        bm = (batch_ref[...] == lax.broadcasted_iota(jnp.int32, (NG, 1), 0))
        p_ref[...] = jnp.dot(bm.astype(jnp.float32), h_ref[...],
                             preferred_element_type=jnp.float32,
                             precision=lax.Precision.HIGHEST)

    return pl.pallas_call(
        body, out_shape=jax.ShapeDtypeStruct((NG, d), jnp.float32),
    )(h, batch2d)


def _proj2(h, Wln, Wrn, dnp):
    def body(h_ref, wl_ref, wr_ref, xl_ref, xr_ref):
        h = h_ref[...]
        xl = jnp.dot(h, wl_ref[...], preferred_element_type=jnp.float32)
        xr = jnp.dot(h, wr_ref[...], preferred_element_type=jnp.float32)
        dn = xl.shape[1]
        if dnp > dn:
            z = jnp.zeros((N, dnp - dn), jnp.float32)
            xl = jnp.concatenate([xl, z], axis=1)
            xr = jnp.concatenate([xr, z], axis=1)
        xl_ref[...] = xl
        xr_ref[...] = xr

    return pl.pallas_call(
        body,
        out_shape=(jax.ShapeDtypeStruct((N, dnp), jnp.float32),
                   jax.ShapeDtypeStruct((N, dnp), jnp.float32)),
    )(h, Wln, Wrn)


def _head_body(p1_ref, p2_ref, p3_ref, p4_ref, w1_ref, b1_ref, g5_ref,
               be5_ref, w2_ref, b2_ref, sig_ref, lin_ref):
    hcat = jnp.concatenate(
        [p1_ref[...], p2_ref[...], p3_ref[...], p4_ref[...]], axis=1)
    h = jnp.maximum(hcat @ w1_ref[...] + b1_ref[...], 0.0)
    mu = jnp.mean(h, axis=0, keepdims=True)
    var = jnp.mean((h - mu) ** 2, axis=0, keepdims=True)
    h = (h - mu) * lax.rsqrt(var + 1e-5) * g5_ref[...] + be5_ref[...]
    o = h @ w2_ref[...] + b2_ref[...]
    lin_ref[...] = o
    sig_ref[...] = jax.nn.sigmoid(o)


def _head(p1, p2, p3, p4, lin1_W, lin1_b, g5, be5, lin2_W, lin2_b):
    dout = lin2_W.shape[1]
    return pl.pallas_call(
        _head_body,
        out_shape=(jax.ShapeDtypeStruct((NG, dout), jnp.float32),
                   jax.ShapeDtypeStruct((NG, dout), jnp.float32)),
    )(p1, p2, p3, p4, lin1_W, lin1_b[None, :], g5[None, :], be5[None, :],
      lin2_W, lin2_b[None, :])


def kernel(x, edge_index, batch, train, Wl1, Wr1, att1, b1, g1, be1, Wl2, Wr2,
           att2, b2, g2, be2, Wl3, Wr3, att3, b3, g3, be3, Wl4, Wr4, att4, b4,
           g4, be4, lin1_W, lin1_b, g5, be5, lin2_W, lin2_b):
    loop = jnp.arange(N, dtype=jnp.int32)
    pad = EP - E0 - N
    src = jnp.concatenate(
        [edge_index[0], loop, jnp.zeros((pad,), jnp.int32)])
    dst = jnp.concatenate(
        [edge_index[1], loop, jnp.full((pad,), N, jnp.int32)])
    src2d = src.reshape(EB, 128)
    d0 = jnp.where(dst < H, dst, H)
    d1 = jnp.where((dst >= H) & (dst < N), dst - H, H)
    dloc1 = jnp.stack([d0, d1]).reshape(2, EB, 128)
    dloc2 = jnp.stack([dst, dst]).reshape(2, EB, 128)
    dgat = jnp.where(dst < N, dst, 0).reshape(EB, 128)
    batch2d = batch.reshape(1, N)
    att1r = lax.reduce_precision(att1, 8, 7)
    att2r = lax.reduce_precision(att2, 8, 7)
    att3p = jnp.concatenate([lax.reduce_precision(att3, 8, 7),
                             jnp.zeros((64,), jnp.float32)])
    att4p = jnp.concatenate([lax.reduce_precision(att4, 8, 7),
                             jnp.zeros((96,), jnp.float32)])

    xlh1, xrh1 = _mm2h(x, Wl1, Wr1)
    tA, wtab = _SC1A(xlh1, xrh1, src2d, dloc1, dgat, att1r)
    tB = _SC1B(xlh1, src2d, dloc2, wtab)
    h1 = _hk1(tA, tB, b1, g1, be1)
    p1 = _pool(h1, batch2d)
    xl2, xr2 = _proj2(h1, Wl2, Wr2, 128)
    t2n, t2d = _SC2(xl2, xr2, src2d, dloc1, dgat, att2r)
    h2 = _hk2(t2n, t2d, b2, g2, be2)
    p2 = _pool(h2, batch2d)
    xl3, xr3 = _proj2(h2, Wl3, Wr3, 128)
    t3 = _SC3(xl3, xr3, src2d, dloc2, dgat, att3p)
    h3 = _hk34(t3, b3, g3, be3, 64)
    p3 = _pool(h3, batch2d)
    xl4, xr4 = _proj2(h3, Wl4, Wr4, 128)
    t4 = _SC4(xl4, xr4, src2d, dloc2, dgat, att4p)
    h4 = _hk34(t4, b4, g4, be4, 32)
    p4 = _pool(h4, batch2d)
    return _head(p1, p2, p3, p4, lin1_W, lin1_b, g5, be5, lin2_W, lin2_b)
